# baseline (speedup 1.0000x reference)
import functools

import jax
import jax.numpy as jnp
from jax import lax
from jax.experimental import pallas as pl
from jax.experimental.pallas import tpu as pltpu
from jax.experimental.pallas import tpu_sc as plsc

_LBASE = 256
_LPA = 128
_NUM_LS = 60
_NUM_PA = 6
_NF = 16
_GROUP = 6561
_B = 16384
_TB = 256


_NC = 2
_NS = 16
_NW = _NC * _NS
_CH = 64
_NSPLIT = 4


def _sc_acc(acc, stg, rows, groups):
    @plsc.parallel_loop(0, rows, 1, unroll=1)
    def row_body(r):
        for g in range(groups):
            plsc.addupdate(acc.at[r, pl.ds(g * 16, 16)],
                           stg[r, pl.ds(g * 16, 16)])


def _sc_acc_pair(acc, stga, stgb, rows, groups):
    @plsc.parallel_loop(0, rows, 1, unroll=1)
    def row_body(r):
        for g in range(groups):
            sl = pl.ds(g * 16, 16)
            plsc.addupdate(acc.at[r, sl], stga[r, sl] + stgb[r, sl])


def _sc_table_pass_vadd(table_hbm, idx_hbm, out_hbm, idxf, acc, stg,
                        rows, sems, wid, nchunk):
    groups = 8

    def chunk_body(c, carry):
        cg = wid * nchunk + c
        iw = {}
        for f in range(_NF):
            iw[f] = pltpu.async_copy(idx_hbm.at[cg, f], idxf[f], sems[4])
        for f in range(_NF):
            iw[f].wait()
        w = {}
        w[0] = pltpu.async_copy(table_hbm.at[idxf[0]], acc, sems[4])
        for f in (1, 2, 3, 4):
            s = (f - 1) % 4
            w[f] = pltpu.async_copy(table_hbm.at[idxf[f]], stg.at[s],
                                    sems[s])
        w[0].wait()
        for j in range(7):
            fa, fb = 2 * j + 1, 2 * j + 2
            sa, sb = (fa - 1) % 4, (fb - 1) % 4
            w[fa].wait()
            w[fb].wait()
            _sc_acc_pair(acc, stg.at[sa], stg.at[sb], rows, groups)
            for nf in (fa + 4, fb + 4):
                if nf < _NF:
                    ns = (nf - 1) % 4
                    w[nf] = pltpu.async_copy(table_hbm.at[idxf[nf]],
                                             stg.at[ns], sems[ns])
        w[_NF - 1].wait()
        _sc_acc(acc, stg.at[(_NF - 2) % 4], rows, groups)
        pltpu.sync_copy(acc, out_hbm.at[pl.ds(cg * rows, rows)])
        return carry

    lax.fori_loop(0, nchunk, chunk_body, 0)


def _sc_gather(idxb_c, idxp_c, base2, pa_table, nb):
    mesh = plsc.VectorSubcoreMesh(core_axis_name="c", subcore_axis_name="s")
    nchunk = nb // _CH // _NW

    @functools.partial(
        pl.kernel,
        out_type=(jax.ShapeDtypeStruct((2 * nb, _LPA), jnp.float32),
                  jax.ShapeDtypeStruct((nb, _LPA), jnp.float32)),
        mesh=mesh,
        scratch_types=[pltpu.VMEM((2 * _CH,), jnp.int32)] * _NF
          + [pltpu.VMEM((_CH,), jnp.int32)] * _NF
          + [pltpu.SemaphoreType.DMA] * 5,
    )
    def sc_kernel(idxb_hbm, idxp_hbm, base_hbm, pa_hbm, xb_out, pa_out,
                  *rest):
        idxfb = rest[:_NF]
        idxfp = rest[_NF:2 * _NF]
        sems = rest[2 * _NF:]
        cid = lax.axis_index("c")
        sid = lax.axis_index("s")
        wid = sid * _NC + cid

        def base_pass(accb, stgb):
            _sc_table_pass_vadd(base_hbm, idxb_hbm, xb_out, idxfb,
                                accb, stgb, 2 * _CH, sems, wid, nchunk)

        pl.run_scoped(base_pass,
                      pltpu.VMEM((2 * _CH, _LPA), jnp.float32),
                      pltpu.VMEM((4, 2 * _CH, _LPA), jnp.float32))

        def pa_pass(accp, stgp):
            _sc_table_pass_vadd(pa_hbm, idxp_hbm, pa_out, idxfp,
                                accp, stgp, _CH, sems, wid, nchunk)

        pl.run_scoped(pa_pass,
                      pltpu.VMEM((_CH, _LPA), jnp.float32),
                      pltpu.VMEM((4, _CH, _LPA), jnp.float32))

    return sc_kernel(idxb_c, idxp_c, base2, pa_table)


def _dense_body(ply_ref, xb_ref, pa_ref, mob_ref, bb_ref, pb_ref,
                paw_ref, pab_ref, w1a_ref, w1b_ref, w1c_ref, b1_ref,
                w2a_ref, w2b_ref, b2_ref, woa_ref, wob_ref, woc_ref, ob_ref,
                out_ref):
    f32 = jnp.float32
    bf16 = jnp.bfloat16
    ls = ply_ref[...]
    dot_t = lambda a, b: lax.dot_general(
        a.astype(bf16), b.astype(bf16), (((1,), (1,)), ((), ())),
        preferred_element_type=f32)
    dot_n = lambda a, b: lax.dot_general(
        a, b, (((1,), (0,)), ((), ())), preferred_element_type=f32)

    xb_full = jnp.clip(xb_ref[...] + bb_ref[...], 0.0, 1.0)
    xb = xb_full[:, :128] * xb_full[:, 128:]
    pa_act = jnp.clip(pa_ref[...] + pb_ref[...], 0.0, 1.0)
    pa_act = pa_act * pa_act

    ypa = dot_t(pa_act, paw_ref[...]) + pab_ref[...]
    j768 = lax.broadcasted_iota(jnp.int32, (_TB, 768), 1)
    ypa = jnp.where((j768 // 128) == (ls // 10), ypa, 0.0)
    spa = (lax.broadcasted_iota(jnp.int32, (768, 128), 0) % 128
           == lax.broadcasted_iota(jnp.int32, (768, 128), 1)).astype(f32)
    x_pa = dot_n(ypa, spa)

    mob = jnp.minimum(mob_ref[...] * (7.0 / 255.0), 1.0)

    y1 = (dot_t(xb, w1a_ref[...]) + dot_t(x_pa, w1b_ref[...])
          + mob * w1c_ref[...] + b1_ref[...])
    j960 = lax.broadcasted_iota(jnp.int32, (_TB, 960), 1)
    y1 = jnp.where((j960 // 16) == ls, y1, 0.0)
    s1 = (lax.broadcasted_iota(jnp.int32, (960, 16), 0) % 16
          == lax.broadcasted_iota(jnp.int32, (960, 16), 1)).astype(f32)
    h = dot_n(y1, s1)
    ha = jnp.clip(h * h, 0.0, 1.0)
    hb = jnp.clip(h, 0.0, 1.0)

    y2 = dot_t(ha, w2a_ref[...]) + dot_t(hb, w2b_ref[...]) + b2_ref[...]
    j3840 = lax.broadcasted_iota(jnp.int32, (_TB, 3840), 1)
    y2 = jnp.where((j3840 // 64) == ls, y2, 0.0)
    s2 = (lax.broadcasted_iota(jnp.int32, (3840, 64), 0) % 64
          == lax.broadcasted_iota(jnp.int32, (3840, 64), 1)).astype(f32)
    l2x = dot_n(y2, s2)
    l2x = jnp.clip(l2x, 0.0, 1.0)
    l2x = l2x * l2x

    yo = (dot_t(l2x, woa_ref[...]) + dot_t(xb, wob_ref[...])
          + dot_t(x_pa, woc_ref[...]) + ob_ref[...])
    j60 = lax.broadcasted_iota(jnp.int32, (_TB, 60), 1)
    yo = jnp.where(j60 == ls, yo, 0.0)
    out_ref[...] = jnp.sum(yo, axis=1, keepdims=True)


def _dense_forward(ply2, xb_raw, pa_raw, mobility, base_bias, pa_bias,
                   paw, pab, w1a, w1b, w1c, b1, w2a, w2b, b2,
                   woa, wob, woc, ob, nb):
    grid = (nb // _TB,)
    row_spec = lambda w: pl.BlockSpec((_TB, w), lambda i: (i, 0))
    full_spec = lambda s: pl.BlockSpec(s, lambda i: (0, 0))
    return pl.pallas_call(
        _dense_body,
        grid=grid,
        in_specs=[
            row_spec(1),
            row_spec(_LBASE),
            row_spec(_LPA),
            row_spec(1),
            full_spec((1, _LBASE)),
            full_spec((1, _LPA)),
            full_spec((768, 128)),
            full_spec((1, 768)),
            full_spec((960, 128)),
            full_spec((960, 128)),
            full_spec((1, 960)),
            full_spec((1, 960)),
            full_spec((3840, 16)),
            full_spec((3840, 16)),
            full_spec((1, 3840)),
            full_spec((60, 64)),
            full_spec((60, 128)),
            full_spec((60, 128)),
            full_spec((1, 60)),
        ],
        out_specs=row_spec(1),
        out_shape=jax.ShapeDtypeStruct((nb, 1), jnp.float32),
    )(ply2, xb_raw, pa_raw, mobility, base_bias, pa_bias,
      paw, pab, w1a, w1b, w1c, b1, w2a, w2b, b2, woa, wob, woc, ob)


def kernel(feature_indices, mobility, ply, base_table, base_bias, pa_table,
           pa_bias, pa_W, pa_b, l1_W, l1_b, l2_W, l2_b, out_W, out_b):
    offsets = (jnp.arange(_NF, dtype=jnp.int32) * _GROUP)[None, :]
    idx = feature_indices + offsets

    idx_c = idx.T.reshape(_NF, _B // _CH, _CH).transpose(1, 0, 2)
    idxb_c = jnp.stack([2 * idx_c, 2 * idx_c + 1],
                       axis=-1).reshape(_B // _CH, _NF, 2 * _CH)
    base2 = base_table.reshape(-1, _LPA)

    w1 = l1_W.reshape(_NUM_LS * 16, 257)
    w1a, w1b = w1[:, :128], w1[:, 128:256]
    w1c = w1[:, 256].reshape(1, -1)
    b1 = l1_b.reshape(1, -1)
    paw = pa_W.reshape(_NUM_PA * _LPA, _LPA)
    pab = pa_b.reshape(1, -1)
    w2 = l2_W.reshape(_NUM_LS * 64, 32)
    w2a, w2b = w2[:, :16], w2[:, 16:]
    b2 = l2_b.reshape(1, -1)
    wo = out_W.reshape(_NUM_LS, 320)
    woa, wob, woc = wo[:, :64], wo[:, 64:192], wo[:, 192:]
    ob = out_b.reshape(1, -1)

    nb = _B // _NSPLIT
    ply2 = ply.reshape(_B, 1)
    outs = []
    for s in range(_NSPLIT):
        lo = s * nb
        xb2, pa_raw = _sc_gather(
            idxb_c[lo // _CH:(lo + nb) // _CH],
            idx_c[lo // _CH:(lo + nb) // _CH], base2, pa_table, nb)
        xb_raw = xb2.reshape(nb, _LBASE)
        outs.append(_dense_forward(
            ply2[lo:lo + nb], xb_raw, pa_raw, mobility[lo:lo + nb],
            base_bias.reshape(1, -1), pa_bias.reshape(1, -1),
            paw, pab, w1a, w1b, w1c, b1, w2a, w2b, b2,
            woa, wob, woc, ob, nb))
    return jnp.concatenate(outs, axis=0)


# speedup vs baseline: 1.3157x; 1.3157x over previous
import functools

import jax
import jax.numpy as jnp
from jax import lax
from jax.experimental import pallas as pl
from jax.experimental.pallas import tpu as pltpu
from jax.experimental.pallas import tpu_sc as plsc

_LBASE = 256
_LPA = 128
_NUM_LS = 60
_NUM_PA = 6
_NF = 16
_GROUP = 6561
_B = 16384
_TB = 256


_NC = 2
_NS = 16
_NW = _NC * _NS
_CH = 64
_NSPLIT = 4


def _sc_acc(acc, stg, rows, groups):
    @plsc.parallel_loop(0, rows, 1, unroll=1)
    def row_body(r):
        for g in range(groups):
            plsc.addupdate(acc.at[r, pl.ds(g * 16, 16)],
                           stg[r, pl.ds(g * 16, 16)])


def _sc_acc_pair(acc, stga, stgb, rows, groups):
    @plsc.parallel_loop(0, rows, 1, unroll=1)
    def row_body(r):
        for g in range(groups):
            sl = pl.ds(g * 16, 16)
            plsc.addupdate(acc.at[r, sl], stga[r, sl] + stgb[r, sl])


def _sc_table_pass_vadd(table_hbm, idx_hbm, out_hbm, idx_v, acc, stg,
                        width, sems, wid, nchunk):
    groups = width // 16

    def chunk_body(c, carry):
        cg = wid * nchunk + c
        pltpu.sync_copy(idx_hbm.at[cg], idx_v)
        w = {}
        w[0] = pltpu.async_copy(table_hbm.at[idx_v.at[0]], acc, sems[4])
        for f in (1, 2, 3, 4):
            s = (f - 1) % 4
            w[f] = pltpu.async_copy(table_hbm.at[idx_v.at[f]], stg.at[s],
                                    sems[s])
        w[0].wait()
        for j in range(7):
            fa, fb = 2 * j + 1, 2 * j + 2
            sa, sb = (fa - 1) % 4, (fb - 1) % 4
            w[fa].wait()
            w[fb].wait()
            _sc_acc_pair(acc, stg.at[sa], stg.at[sb], _CH, groups)
            for nf in (fa + 4, fb + 4):
                if nf < _NF:
                    ns = (nf - 1) % 4
                    w[nf] = pltpu.async_copy(table_hbm.at[idx_v.at[nf]],
                                             stg.at[ns], sems[ns])
        w[_NF - 1].wait()
        _sc_acc(acc, stg.at[(_NF - 2) % 4], _CH, groups)
        pltpu.sync_copy(acc, out_hbm.at[pl.ds(cg * _CH, _CH)])
        return carry

    lax.fori_loop(0, nchunk, chunk_body, 0)


def _sc_gather(idx_c, base_table, pa_table, nb):
    mesh = plsc.VectorSubcoreMesh(core_axis_name="c", subcore_axis_name="s")
    nchunk = nb // _CH // _NW

    @functools.partial(
        pl.kernel,
        out_type=(jax.ShapeDtypeStruct((nb, _LBASE), jnp.float32),
                  jax.ShapeDtypeStruct((nb, _LPA), jnp.float32)),
        mesh=mesh,
        scratch_types=[
            pltpu.VMEM((_NF, _CH), jnp.int32),
        ] + [pltpu.SemaphoreType.DMA] * 5,
    )
    def sc_kernel(idx_hbm, base_hbm, pa_hbm, xb_out, pa_out, idx_v, *sems):
        cid = lax.axis_index("c")
        sid = lax.axis_index("s")
        wid = sid * _NC + cid

        def base_pass(accb, stgb):
            _sc_table_pass_vadd(base_hbm, idx_hbm, xb_out, idx_v, accb,
                                stgb, _LBASE, sems, wid, nchunk)

        pl.run_scoped(base_pass,
                      pltpu.VMEM((_CH, _LBASE), jnp.float32),
                      pltpu.VMEM((4, _CH, _LBASE), jnp.float32))

        def pa_pass(accp, stgp):
            _sc_table_pass_vadd(pa_hbm, idx_hbm, pa_out, idx_v, accp,
                                stgp, _LPA, sems, wid, nchunk)

        pl.run_scoped(pa_pass,
                      pltpu.VMEM((_CH, _LPA), jnp.float32),
                      pltpu.VMEM((4, _CH, _LPA), jnp.float32))

    return sc_kernel(idx_c, base_table, pa_table)


def _dense_body(ply_ref, xb_ref, pa_ref, mob_ref, bb_ref, pb_ref,
                paw_ref, pab_ref, w1a_ref, w1b_ref, w1c_ref, b1_ref,
                w2a_ref, w2b_ref, b2_ref, woa_ref, wob_ref, woc_ref, ob_ref,
                out_ref):
    f32 = jnp.float32
    bf16 = jnp.bfloat16
    ls = ply_ref[...]
    dot_t = lambda a, b: lax.dot_general(
        a.astype(bf16), b.astype(bf16), (((1,), (1,)), ((), ())),
        preferred_element_type=f32)
    dot_n = lambda a, b: lax.dot_general(
        a, b, (((1,), (0,)), ((), ())), preferred_element_type=f32)

    xb_full = jnp.clip(xb_ref[...] + bb_ref[...], 0.0, 1.0)
    xb = xb_full[:, :128] * xb_full[:, 128:]
    pa_act = jnp.clip(pa_ref[...] + pb_ref[...], 0.0, 1.0)
    pa_act = pa_act * pa_act

    ypa = dot_t(pa_act, paw_ref[...]) + pab_ref[...]
    j768 = lax.broadcasted_iota(jnp.int32, (_TB, 768), 1)
    ypa = jnp.where((j768 // 128) == (ls // 10), ypa, 0.0)
    spa = (lax.broadcasted_iota(jnp.int32, (768, 128), 0) % 128
           == lax.broadcasted_iota(jnp.int32, (768, 128), 1)).astype(f32)
    x_pa = dot_n(ypa, spa)

    mob = jnp.minimum(mob_ref[...] * (7.0 / 255.0), 1.0)

    y1 = (dot_t(xb, w1a_ref[...]) + dot_t(x_pa, w1b_ref[...])
          + mob * w1c_ref[...] + b1_ref[...])
    j960 = lax.broadcasted_iota(jnp.int32, (_TB, 960), 1)
    y1 = jnp.where((j960 // 16) == ls, y1, 0.0)
    s1 = (lax.broadcasted_iota(jnp.int32, (960, 16), 0) % 16
          == lax.broadcasted_iota(jnp.int32, (960, 16), 1)).astype(f32)
    h = dot_n(y1, s1)
    ha = jnp.clip(h * h, 0.0, 1.0)
    hb = jnp.clip(h, 0.0, 1.0)

    y2 = dot_t(ha, w2a_ref[...]) + dot_t(hb, w2b_ref[...]) + b2_ref[...]
    j3840 = lax.broadcasted_iota(jnp.int32, (_TB, 3840), 1)
    y2 = jnp.where((j3840 // 64) == ls, y2, 0.0)
    s2 = (lax.broadcasted_iota(jnp.int32, (3840, 64), 0) % 64
          == lax.broadcasted_iota(jnp.int32, (3840, 64), 1)).astype(f32)
    l2x = dot_n(y2, s2)
    l2x = jnp.clip(l2x, 0.0, 1.0)
    l2x = l2x * l2x

    yo = (dot_t(l2x, woa_ref[...]) + dot_t(xb, wob_ref[...])
          + dot_t(x_pa, woc_ref[...]) + ob_ref[...])
    j60 = lax.broadcasted_iota(jnp.int32, (_TB, 60), 1)
    yo = jnp.where(j60 == ls, yo, 0.0)
    out_ref[...] = jnp.sum(yo, axis=1, keepdims=True)


def _dense_forward(ply2, xb_raw, pa_raw, mobility, base_bias, pa_bias,
                   paw, pab, w1a, w1b, w1c, b1, w2a, w2b, b2,
                   woa, wob, woc, ob, nb):
    grid = (nb // _TB,)
    row_spec = lambda w: pl.BlockSpec((_TB, w), lambda i: (i, 0))
    full_spec = lambda s: pl.BlockSpec(s, lambda i: (0, 0))
    return pl.pallas_call(
        _dense_body,
        grid=grid,
        in_specs=[
            row_spec(1),
            row_spec(_LBASE),
            row_spec(_LPA),
            row_spec(1),
            full_spec((1, _LBASE)),
            full_spec((1, _LPA)),
            full_spec((768, 128)),
            full_spec((1, 768)),
            full_spec((960, 128)),
            full_spec((960, 128)),
            full_spec((1, 960)),
            full_spec((1, 960)),
            full_spec((3840, 16)),
            full_spec((3840, 16)),
            full_spec((1, 3840)),
            full_spec((60, 64)),
            full_spec((60, 128)),
            full_spec((60, 128)),
            full_spec((1, 60)),
        ],
        out_specs=row_spec(1),
        out_shape=jax.ShapeDtypeStruct((nb, 1), jnp.float32),
    )(ply2, xb_raw, pa_raw, mobility, base_bias, pa_bias,
      paw, pab, w1a, w1b, w1c, b1, w2a, w2b, b2, woa, wob, woc, ob)


def kernel(feature_indices, mobility, ply, base_table, base_bias, pa_table,
           pa_bias, pa_W, pa_b, l1_W, l1_b, l2_W, l2_b, out_W, out_b):
    offsets = (jnp.arange(_NF, dtype=jnp.int32) * _GROUP)[None, :]
    idx = feature_indices + offsets

    idx_c = idx.T.reshape(_NF, _B // _CH, _CH).transpose(1, 0, 2)

    w1 = l1_W.reshape(_NUM_LS * 16, 257)
    w1a, w1b = w1[:, :128], w1[:, 128:256]
    w1c = w1[:, 256].reshape(1, -1)
    b1 = l1_b.reshape(1, -1)
    paw = pa_W.reshape(_NUM_PA * _LPA, _LPA)
    pab = pa_b.reshape(1, -1)
    w2 = l2_W.reshape(_NUM_LS * 64, 32)
    w2a, w2b = w2[:, :16], w2[:, 16:]
    b2 = l2_b.reshape(1, -1)
    wo = out_W.reshape(_NUM_LS, 320)
    woa, wob, woc = wo[:, :64], wo[:, 64:192], wo[:, 192:]
    ob = out_b.reshape(1, -1)

    nb = _B // _NSPLIT
    ply2 = ply.reshape(_B, 1)
    outs = []
    for s in range(_NSPLIT):
        lo = s * nb
        xb_raw, pa_raw = _sc_gather(
            idx_c[lo // _CH:(lo + nb) // _CH], base_table, pa_table, nb)
        outs.append(_dense_forward(
            ply2[lo:lo + nb], xb_raw, pa_raw, mobility[lo:lo + nb],
            base_bias.reshape(1, -1), pa_bias.reshape(1, -1),
            paw, pab, w1a, w1b, w1c, b1, w2a, w2b, b2,
            woa, wob, woc, ob, nb))
    return jnp.concatenate(outs, axis=0)
